# Initial kernel scaffold; baseline (speedup 1.0000x reference)
import jax
import jax.numpy as jnp
from jax.experimental import pallas as pl


def kernel(x, edge_index, W1a, b1a, g1a, be1a, W2a, b2a, g2a, be2a, W1b, b1b, g1b, be1b, W2b, b2b, g2b, be2b, Wmu, bmu, Wls, bls):
    raise NotImplementedError("write your pallas kernel here")


# speedup vs baseline: 4.4917x; 4.4917x over previous
import jax
import jax.numpy as jnp
from jax import lax
from jax.experimental import pallas as pl
from jax.experimental.pallas import tpu as pltpu
from jax.experimental.pallas import tpu_sc as plsc

N = 10000
D = 128
E = 320000
NC = 2
NS = 16
NW = NC * NS
EPW = E // NW
K = 80
NCHUNK = EPW // K
RPS = 624
REM = N - NS * RPS
ZROWS = 104
BN_EPS = 1e-5


def _sc_agg_body(x_hbm, src_hbm, dst_hbm, out_hbm,
                 src_v, dst_v, rows_v, zbuf_v, agg_sh, sem):
    c = lax.axis_index("c")
    s = lax.axis_index("s")
    wid = c * NS + s
    base = wid * EPW

    z = jnp.zeros((16,), jnp.float32)

    def zrow(i, carry):
        def zcol(j, carry2):
            zbuf_v[i, pl.ds(j * 16, 16)] = z
            return carry2
        return lax.fori_loop(0, D // 16, zcol, carry)

    lax.fori_loop(0, ZROWS, zrow, 0)

    def zcopy(t, carry):
        pltpu.sync_copy(zbuf_v, agg_sh.at[pl.ds(s * RPS + t * ZROWS, ZROWS)])
        return carry

    lax.fori_loop(0, RPS // ZROWS, zcopy, 0)

    @pl.when(s == NS - 1)
    def _zero_rem():
        pltpu.sync_copy(zbuf_v.at[pl.ds(0, REM)],
                        agg_sh.at[pl.ds(NS * RPS, REM)])

    plsc.subcore_barrier()

    def chunk(j, carry):
        pltpu.sync_copy(src_hbm.at[pl.ds(base + j * K, K)], src_v)
        pltpu.sync_copy(dst_hbm.at[pl.ds(base + j * K, K)], dst_v)
        pltpu.async_copy(x_hbm.at[src_v], rows_v, sem).wait()
        pltpu.sync_copy(rows_v, agg_sh.at[dst_v], add=True)
        return carry

    lax.fori_loop(0, NCHUNK, chunk, 0)
    plsc.subcore_barrier()

    pltpu.sync_copy(agg_sh.at[pl.ds(s * RPS, RPS)],
                    out_hbm.at[c, pl.ds(s * RPS, RPS)])

    @pl.when(s == NS - 1)
    def _write_rem():
        pltpu.sync_copy(agg_sh.at[pl.ds(NS * RPS, REM)],
                        out_hbm.at[c, pl.ds(NS * RPS, REM)])


def _sc_agg(x, src_r, dst_r):
    mesh = plsc.VectorSubcoreMesh(core_axis_name="c", subcore_axis_name="s")
    return pl.kernel(
        _sc_agg_body,
        out_type=jax.ShapeDtypeStruct((NC, N, D), jnp.float32),
        mesh=mesh,
        scratch_types=[
            pltpu.VMEM((K,), jnp.int32),
            pltpu.VMEM((K,), jnp.int32),
            pltpu.VMEM((K, D), jnp.float32),
            pltpu.VMEM((ZROWS, D), jnp.float32),
            pltpu.VMEM_SHARED((N, D), jnp.float32),
            pltpu.SemaphoreType.DMA,
        ],
    )(x, src_r, dst_r)


def _bn_relu(h, g, be):
    m = jnp.mean(h, axis=0, keepdims=True)
    v = jnp.mean((h - m) ** 2, axis=0, keepdims=True)
    return jnp.maximum((h - m) * lax.rsqrt(v + BN_EPS) * g + be, 0.0)


def _mlp1_body(x_ref, a0, a1, w1, b1, g1, be1, w2, b2, g2, be2, out_ref):
    h = x_ref[...] + a0[...] + a1[...]
    h = jnp.dot(h, w1[...], preferred_element_type=jnp.float32) + b1[...]
    h = _bn_relu(h, g1[...], be1[...])
    h = jnp.dot(h, w2[...], preferred_element_type=jnp.float32) + b2[...]
    out_ref[...] = _bn_relu(h, g2[...], be2[...])


def _mlp2_body(x_ref, a0, a1, w1, b1, g1, be1, w2, b2, g2, be2, wh, bh,
               out_ref):
    h = x_ref[...] + a0[...] + a1[...]
    h = jnp.dot(h, w1[...], preferred_element_type=jnp.float32) + b1[...]
    h = _bn_relu(h, g1[...], be1[...])
    h = jnp.dot(h, w2[...], preferred_element_type=jnp.float32) + b2[...]
    h = _bn_relu(h, g2[...], be2[...])
    out_ref[...] = jnp.dot(h, wh[...], preferred_element_type=jnp.float32) + bh[...]


def _mlp1(x, a0, a1, w1, b1, g1, be1, w2, b2, g2, be2):
    return pl.pallas_call(
        _mlp1_body,
        out_shape=jax.ShapeDtypeStruct((N, D), jnp.float32),
    )(x, a0, a1, w1, b1, g1, be1, w2, b2, g2, be2)


def _mlp2(x, a0, a1, w1, b1, g1, be1, w2, b2, g2, be2, wh, bh):
    return pl.pallas_call(
        _mlp2_body,
        out_shape=jax.ShapeDtypeStruct((N, 2 * 64), jnp.float32),
    )(x, a0, a1, w1, b1, g1, be1, w2, b2, g2, be2, wh, bh)


def kernel(x, edge_index, W1a, b1a, g1a, be1a, W2a, b2a, g2a, be2a,
           W1b, b1b, g1b, be1b, W2b, b2b, g2b, be2b, Wmu, bmu, Wls, bls):
    src = edge_index[0].astype(jnp.int32)
    dst = edge_index[1].astype(jnp.int32)

    r = lambda v: v.reshape(1, -1)
    wh = jnp.concatenate([Wmu, Wls], axis=1)
    bh = jnp.concatenate([bmu, bls], axis=0).reshape(1, -1)

    agg1 = _sc_agg(x, src, dst)
    h1 = _mlp1(x, agg1[0], agg1[1], W1a, r(b1a), r(g1a), r(be1a),
               W2a, r(b2a), r(g2a), r(be2a))
    agg2 = _sc_agg(h1, src, dst)
    heads = _mlp2(h1, agg2[0], agg2[1], W1b, r(b1b), r(g1b), r(be1b),
                  W2b, r(b2b), r(g2b), r(be2b), wh, bh)
    return (heads[:, :64], heads[:, 64:])


# speedup vs baseline: 7.0081x; 1.5602x over previous
import jax
import jax.numpy as jnp
from jax import lax
from jax.experimental import pallas as pl
from jax.experimental.pallas import tpu as pltpu
from jax.experimental.pallas import tpu_sc as plsc

N = 10000
D = 128
E = 320000
NC = 2
NS = 16
NW = NC * NS
EPW = E // NW
K = 80
NCHUNK = EPW // K
RPS = 624
REM = N - NS * RPS
ZROWS = 104
BN_EPS = 1e-5


def _sc_agg_body(x_hbm, src_hbm, dst_hbm, out_hbm,
                 src0, dst0, src1, dst1, rows0, rows1, zbuf_v, agg_sh,
                 sem0, sem1):
    c = lax.axis_index("c")
    s = lax.axis_index("s")
    wid = c * NS + s
    base = wid * EPW

    z = jnp.zeros((16,), jnp.float32)

    def zrow(i, carry):
        def zcol(j, carry2):
            zbuf_v[i, pl.ds(j * 16, 16)] = z
            return carry2
        return lax.fori_loop(0, D // 16, zcol, carry)

    lax.fori_loop(0, ZROWS, zrow, 0)

    def zcopy(t, carry):
        pltpu.sync_copy(zbuf_v, agg_sh.at[pl.ds(s * RPS + t * ZROWS, ZROWS)])
        return carry

    lax.fori_loop(0, RPS // ZROWS, zcopy, 0)

    @pl.when(s == NS - 1)
    def _zero_rem():
        pltpu.sync_copy(zbuf_v.at[pl.ds(0, REM)],
                        agg_sh.at[pl.ds(NS * RPS, REM)])

    plsc.subcore_barrier()

    def stage_and_fire(j, src_b, dst_b, rows_b, sem_b):
        pltpu.sync_copy(src_hbm.at[pl.ds(base + j * K, K)], src_b)
        pltpu.sync_copy(dst_hbm.at[pl.ds(base + j * K, K)], dst_b)
        pltpu.async_copy(x_hbm.at[src_b], rows_b, sem_b)

    stage_and_fire(0, src0, dst0, rows0, sem0)
    stage_and_fire(1, src1, dst1, rows1, sem1)

    def chunk_pair(t, carry):
        pltpu.make_async_copy(x_hbm.at[src0], rows0, sem0).wait()
        pltpu.sync_copy(rows0, agg_sh.at[dst0], add=True)
        stage_and_fire(2 * t + 2, src0, dst0, rows0, sem0)

        pltpu.make_async_copy(x_hbm.at[src1], rows1, sem1).wait()
        pltpu.sync_copy(rows1, agg_sh.at[dst1], add=True)

        @pl.when(2 * t + 3 < NCHUNK)
        def _fire_odd():
            stage_and_fire(2 * t + 3, src1, dst1, rows1, sem1)

        return carry

    lax.fori_loop(0, (NCHUNK - 1) // 2, chunk_pair, 0)
    pltpu.make_async_copy(x_hbm.at[src0], rows0, sem0).wait()
    pltpu.sync_copy(rows0, agg_sh.at[dst0], add=True)
    plsc.subcore_barrier()

    pltpu.sync_copy(agg_sh.at[pl.ds(s * RPS, RPS)],
                    out_hbm.at[c, pl.ds(s * RPS, RPS)])

    @pl.when(s == NS - 1)
    def _write_rem():
        pltpu.sync_copy(agg_sh.at[pl.ds(NS * RPS, REM)],
                        out_hbm.at[c, pl.ds(NS * RPS, REM)])


def _sc_agg(x, src_r, dst_r):
    mesh = plsc.VectorSubcoreMesh(core_axis_name="c", subcore_axis_name="s")
    return pl.kernel(
        _sc_agg_body,
        out_type=jax.ShapeDtypeStruct((NC, N, D), jnp.float32),
        mesh=mesh,
        scratch_types=[
            pltpu.VMEM((K,), jnp.int32),
            pltpu.VMEM((K,), jnp.int32),
            pltpu.VMEM((K,), jnp.int32),
            pltpu.VMEM((K,), jnp.int32),
            pltpu.VMEM((K, D), jnp.float32),
            pltpu.VMEM((K, D), jnp.float32),
            pltpu.VMEM((ZROWS, D), jnp.float32),
            pltpu.VMEM_SHARED((N, D), jnp.float32),
            pltpu.SemaphoreType.DMA,
            pltpu.SemaphoreType.DMA,
        ],
    )(x, src_r, dst_r)


def _bn_relu(h, g, be):
    m = jnp.mean(h, axis=0, keepdims=True)
    v = jnp.mean((h - m) ** 2, axis=0, keepdims=True)
    return jnp.maximum((h - m) * lax.rsqrt(v + BN_EPS) * g + be, 0.0)


def _mlp1_body(x_ref, a0, a1, w1, b1, g1, be1, w2, b2, g2, be2, out_ref):
    h = x_ref[...] + a0[...] + a1[...]
    h = jnp.dot(h, w1[...], preferred_element_type=jnp.float32) + b1[...]
    h = _bn_relu(h, g1[...], be1[...])
    h = jnp.dot(h, w2[...], preferred_element_type=jnp.float32) + b2[...]
    out_ref[...] = _bn_relu(h, g2[...], be2[...])


def _mlp2_body(x_ref, a0, a1, w1, b1, g1, be1, w2, b2, g2, be2, wh, bh,
               out_ref):
    h = x_ref[...] + a0[...] + a1[...]
    h = jnp.dot(h, w1[...], preferred_element_type=jnp.float32) + b1[...]
    h = _bn_relu(h, g1[...], be1[...])
    h = jnp.dot(h, w2[...], preferred_element_type=jnp.float32) + b2[...]
    h = _bn_relu(h, g2[...], be2[...])
    out_ref[...] = jnp.dot(h, wh[...], preferred_element_type=jnp.float32) + bh[...]


def _mlp1(x, a0, a1, w1, b1, g1, be1, w2, b2, g2, be2):
    return pl.pallas_call(
        _mlp1_body,
        out_shape=jax.ShapeDtypeStruct((N, D), jnp.float32),
    )(x, a0, a1, w1, b1, g1, be1, w2, b2, g2, be2)


def _mlp2(x, a0, a1, w1, b1, g1, be1, w2, b2, g2, be2, wh, bh):
    return pl.pallas_call(
        _mlp2_body,
        out_shape=jax.ShapeDtypeStruct((N, 2 * 64), jnp.float32),
    )(x, a0, a1, w1, b1, g1, be1, w2, b2, g2, be2, wh, bh)


def kernel(x, edge_index, W1a, b1a, g1a, be1a, W2a, b2a, g2a, be2a,
           W1b, b1b, g1b, be1b, W2b, b2b, g2b, be2b, Wmu, bmu, Wls, bls):
    src = edge_index[0].astype(jnp.int32)
    dst = edge_index[1].astype(jnp.int32)

    r = lambda v: v.reshape(1, -1)
    wh = jnp.concatenate([Wmu, Wls], axis=1)
    bh = jnp.concatenate([bmu, bls], axis=0).reshape(1, -1)

    agg1 = _sc_agg(x, src, dst)
    h1 = _mlp1(x, agg1[0], agg1[1], W1a, r(b1a), r(g1a), r(be1a),
               W2a, r(b2a), r(g2a), r(be2a))
    agg2 = _sc_agg(h1, src, dst)
    heads = _mlp2(h1, agg2[0], agg2[1], W1b, r(b1b), r(g1b), r(be1b),
                  W2b, r(b2b), r(g2b), r(be2b), wh, bh)
    return (heads[:, :64], heads[:, 64:])


# speedup vs baseline: 11.1261x; 1.5876x over previous
import jax
import jax.numpy as jnp
from jax import lax
from jax.experimental import pallas as pl
from jax.experimental.pallas import tpu as pltpu
from jax.experimental.pallas import tpu_sc as plsc

N = 10000
D = 128
E = 320000
NC = 2
NS = 16
NW = NC * NS
EPW = E // NW
K = 80
NCHUNK = EPW // K
RPS = 624
REM = N - NS * RPS
ZROWS = 48
NBUF = 4
PF = 3
BN_EPS = 1e-5


def _sc_agg_body(x_hbm, pk_hbm, out_hbm,
                 pk_v, src_v, dst_v, rows_v, zbuf_v, agg_sh,
                 gs0, gs1, gs2, gs3, ss0, ss1, ss2, ss3,
                 is0, is1, is2, is3):
    gsem = (gs0, gs1, gs2, gs3)
    ssem = (ss0, ss1, ss2, ss3)
    isem = (is0, is1, is2, is3)
    c = lax.axis_index("c")
    s = lax.axis_index("s")
    wid = c * NS + s
    base = wid * EPW

    z = jnp.zeros((16,), jnp.float32)

    def zrow(i, carry):
        def zcol(j, carry2):
            zbuf_v[i, pl.ds(j * 16, 16)] = z
            return carry2
        return lax.fori_loop(0, D // 16, zcol, carry)

    lax.fori_loop(0, ZROWS, zrow, 0)

    def zcopy(t, carry):
        pltpu.sync_copy(zbuf_v, agg_sh.at[pl.ds(s * RPS + t * ZROWS, ZROWS)])
        return carry

    lax.fori_loop(0, RPS // ZROWS, zcopy, 0)

    @pl.when(s == NS - 1)
    def _zero_rem():
        pltpu.sync_copy(zbuf_v.at[pl.ds(0, REM)],
                        agg_sh.at[pl.ds(NS * RPS, REM)])

    plsc.subcore_barrier()

    def fire_pk(j, b):
        pltpu.async_copy(pk_hbm.at[pl.ds(base + j * K, K)], pk_v.at[b],
                         isem[b])

    def wait_pk(b):
        pltpu.make_async_copy(pk_hbm.at[pl.ds(base, K)], pk_v.at[b],
                              isem[b]).wait()

    def decode(b):
        for u in range(K // 16):
            v = pk_v[b, pl.ds(u * 16, 16)]
            src_v[b, pl.ds(u * 16, 16)] = v & 0xFFFF
            dst_v[b, pl.ds(u * 16, 16)] = lax.shift_right_logical(v, 16)

    def fire_gather(b):
        pltpu.async_copy(x_hbm.at[src_v.at[b]], rows_v.at[b], gsem[b])

    def wait_gather(b):
        pltpu.make_async_copy(x_hbm.at[src_v.at[b]], rows_v.at[b],
                              gsem[b]).wait()

    def fire_scatter(b):
        pltpu.async_copy(rows_v.at[b], agg_sh.at[dst_v.at[b]], ssem[b],
                         add=True)

    def wait_scatter(b):
        pltpu.make_async_copy(rows_v.at[b], agg_sh.at[dst_v.at[b]],
                              ssem[b]).wait()

    for jj in range(PF):
        fire_pk(jj, jj)
    for jj in range(PF):
        wait_pk(jj)
        decode(jj)
        fire_gather(jj)

    def quad(t, carry):
        for r in range(NBUF):
            j = NBUF * t + r
            wait_gather(r)
            fire_scatter(r)
            fb = (r + PF) % NBUF
            fj = j + PF

            @pl.when(fj < NCHUNK)
            def _prefetch(fb=fb, fj=fj):
                fire_pk(fj, fb)

                @pl.when(fj >= NBUF)
                def _free_bufs():
                    wait_scatter(fb)

                wait_pk(fb)
                decode(fb)
                fire_gather(fb)

        return carry

    lax.fori_loop(0, NCHUNK // NBUF, quad, 0)
    wait_gather(0)
    fire_scatter(0)
    wait_scatter(1)
    wait_scatter(2)
    wait_scatter(3)
    wait_scatter(0)
    plsc.subcore_barrier()

    pltpu.sync_copy(agg_sh.at[pl.ds(s * RPS, RPS)],
                    out_hbm.at[c, pl.ds(s * RPS, RPS)])

    @pl.when(s == NS - 1)
    def _write_rem():
        pltpu.sync_copy(agg_sh.at[pl.ds(NS * RPS, REM)],
                        out_hbm.at[c, pl.ds(NS * RPS, REM)])


def _sc_agg(x, packed):
    mesh = plsc.VectorSubcoreMesh(core_axis_name="c", subcore_axis_name="s")
    return pl.kernel(
        _sc_agg_body,
        out_type=jax.ShapeDtypeStruct((NC, N, D), jnp.float32),
        mesh=mesh,
        scratch_types=[
            pltpu.VMEM((NBUF, K), jnp.int32),
            pltpu.VMEM((NBUF, K), jnp.int32),
            pltpu.VMEM((NBUF, K), jnp.int32),
            pltpu.VMEM((NBUF, K, D), jnp.float32),
            pltpu.VMEM((ZROWS, D), jnp.float32),
            pltpu.VMEM_SHARED((N, D), jnp.float32),
        ] + [pltpu.SemaphoreType.DMA] * 12,
    )(x, packed)


def _bn_relu(h, g, be):
    m = jnp.mean(h, axis=0, keepdims=True)
    v = jnp.mean((h - m) ** 2, axis=0, keepdims=True)
    return jnp.maximum((h - m) * lax.rsqrt(v + BN_EPS) * g + be, 0.0)


def _mlp1_body(x_ref, a0, a1, w1, b1, g1, be1, w2, b2, g2, be2, out_ref):
    h = x_ref[...] + a0[...] + a1[...]
    h = jnp.dot(h, w1[...], preferred_element_type=jnp.float32) + b1[...]
    h = _bn_relu(h, g1[...], be1[...])
    h = jnp.dot(h, w2[...], preferred_element_type=jnp.float32) + b2[...]
    out_ref[...] = _bn_relu(h, g2[...], be2[...])


def _mlp2_body(x_ref, a0, a1, w1, b1, g1, be1, w2, b2, g2, be2, wh, bh,
               out_ref):
    h = x_ref[...] + a0[...] + a1[...]
    h = jnp.dot(h, w1[...], preferred_element_type=jnp.float32) + b1[...]
    h = _bn_relu(h, g1[...], be1[...])
    h = jnp.dot(h, w2[...], preferred_element_type=jnp.float32) + b2[...]
    h = _bn_relu(h, g2[...], be2[...])
    out_ref[...] = jnp.dot(h, wh[...], preferred_element_type=jnp.float32) + bh[...]


def _mlp1(x, a0, a1, w1, b1, g1, be1, w2, b2, g2, be2):
    return pl.pallas_call(
        _mlp1_body,
        out_shape=jax.ShapeDtypeStruct((N, D), jnp.float32),
    )(x, a0, a1, w1, b1, g1, be1, w2, b2, g2, be2)


def _mlp2(x, a0, a1, w1, b1, g1, be1, w2, b2, g2, be2, wh, bh):
    return pl.pallas_call(
        _mlp2_body,
        out_shape=jax.ShapeDtypeStruct((N, 2 * 64), jnp.float32),
    )(x, a0, a1, w1, b1, g1, be1, w2, b2, g2, be2, wh, bh)


def kernel(x, edge_index, W1a, b1a, g1a, be1a, W2a, b2a, g2a, be2a,
           W1b, b1b, g1b, be1b, W2b, b2b, g2b, be2b, Wmu, bmu, Wls, bls):
    src = edge_index[0].astype(jnp.int32)
    dst = edge_index[1].astype(jnp.int32)
    packed = src | (dst << 16)

    r = lambda v: v.reshape(1, -1)
    wh = jnp.concatenate([Wmu, Wls], axis=1)
    bh = jnp.concatenate([bmu, bls], axis=0).reshape(1, -1)

    agg1 = _sc_agg(x, packed)
    h1 = _mlp1(x, agg1[0], agg1[1], W1a, r(b1a), r(g1a), r(be1a),
               W2a, r(b2a), r(g2a), r(be2a))
    agg2 = _sc_agg(h1, packed)
    heads = _mlp2(h1, agg2[0], agg2[1], W1b, r(b1b), r(g1b), r(be1b),
                  W2b, r(b2b), r(g2b), r(be2b), wh, bh)
    return (heads[:, :64], heads[:, 64:])


# speedup vs baseline: 11.2274x; 1.0091x over previous
import jax
import jax.numpy as jnp
from jax import lax
from jax.experimental import pallas as pl
from jax.experimental.pallas import tpu as pltpu
from jax.experimental.pallas import tpu_sc as plsc

N = 10000
D = 128
E = 320000
NC = 2
NS = 16
NW = NC * NS
EPW = E // NW
K = 80
NCHUNK = EPW // K
RPS = 624
REM = N - NS * RPS
ZROWS = 48
NBUF = 4
PF = 3
BN_EPS = 1e-5


def _sc_agg_body(x_hbm, pk_hbm, out_hbm,
                 pk_v, src_v, dst_v, rows_v, zbuf_v, agg_sh,
                 gs0, gs1, gs2, gs3, ss0, ss1, ss2, ss3,
                 is0, is1, is2, is3):
    gsem = (gs0, gs1, gs2, gs3)
    ssem = (ss0, ss1, ss2, ss3)
    isem = (is0, is1, is2, is3)
    c = lax.axis_index("c")
    s = lax.axis_index("s")
    wid = c * NS + s
    base = wid * EPW

    z = jnp.zeros((16,), jnp.float32)

    def zrow(i, carry):
        def zcol(j, carry2):
            zbuf_v[i, pl.ds(j * 16, 16)] = z
            return carry2
        return lax.fori_loop(0, D // 16, zcol, carry)

    lax.fori_loop(0, ZROWS, zrow, 0)

    def zcopy(t, carry):
        pltpu.sync_copy(zbuf_v, agg_sh.at[pl.ds(s * RPS + t * ZROWS, ZROWS)])
        return carry

    lax.fori_loop(0, RPS // ZROWS, zcopy, 0)

    @pl.when(s == NS - 1)
    def _zero_rem():
        pltpu.sync_copy(zbuf_v.at[pl.ds(0, REM)],
                        agg_sh.at[pl.ds(NS * RPS, REM)])

    plsc.subcore_barrier()

    def fire_pk(j, b):
        pltpu.async_copy(pk_hbm.at[pl.ds(base + j * K, K)], pk_v.at[b],
                         isem[b])

    def wait_pk(b):
        pltpu.make_async_copy(pk_hbm.at[pl.ds(base, K)], pk_v.at[b],
                              isem[b]).wait()

    def decode(b):
        for u in range(K // 16):
            v = pk_v[b, pl.ds(u * 16, 16)]
            src_v[b, pl.ds(u * 16, 16)] = v & 0xFFFF
            dst_v[b, pl.ds(u * 16, 16)] = lax.shift_right_logical(v, 16)

    def fire_gather(b):
        pltpu.async_copy(x_hbm.at[src_v.at[b]], rows_v.at[b], gsem[b])

    def wait_gather(b):
        pltpu.make_async_copy(x_hbm.at[src_v.at[b]], rows_v.at[b],
                              gsem[b]).wait()

    def fire_scatter(b):
        pltpu.async_copy(rows_v.at[b], agg_sh.at[dst_v.at[b]], ssem[b],
                         add=True)

    def wait_scatter(b):
        pltpu.make_async_copy(rows_v.at[b], agg_sh.at[dst_v.at[b]],
                              ssem[b]).wait()

    for jj in range(PF):
        fire_pk(jj, jj)
    for jj in range(PF):
        wait_pk(jj)
        decode(jj)
        fire_gather(jj)

    def quad(t, carry):
        for r in range(NBUF):
            j = NBUF * t + r
            wait_gather(r)
            fire_scatter(r)
            fb = (r + PF) % NBUF
            fj = j + PF

            @pl.when(fj < NCHUNK)
            def _prefetch(fb=fb, fj=fj):
                fire_pk(fj, fb)

                @pl.when(fj >= NBUF)
                def _free_bufs():
                    wait_scatter(fb)

                wait_pk(fb)
                decode(fb)
                fire_gather(fb)

        return carry

    lax.fori_loop(0, NCHUNK // NBUF, quad, 0)
    wait_gather(0)
    fire_scatter(0)
    wait_scatter(1)
    wait_scatter(2)
    wait_scatter(3)
    wait_scatter(0)
    plsc.subcore_barrier()

    pltpu.sync_copy(agg_sh.at[pl.ds(s * RPS, RPS)],
                    out_hbm.at[c, pl.ds(s * RPS, RPS)])

    @pl.when(s == NS - 1)
    def _write_rem():
        pltpu.sync_copy(agg_sh.at[pl.ds(NS * RPS, REM)],
                        out_hbm.at[c, pl.ds(NS * RPS, REM)])


def _sc_agg(x, packed):
    mesh = plsc.VectorSubcoreMesh(core_axis_name="c", subcore_axis_name="s")
    return pl.kernel(
        _sc_agg_body,
        out_type=jax.ShapeDtypeStruct((NC, N, D), jnp.float32),
        mesh=mesh,
        scratch_types=[
            pltpu.VMEM((NBUF, K), jnp.int32),
            pltpu.VMEM((NBUF, K), jnp.int32),
            pltpu.VMEM((NBUF, K), jnp.int32),
            pltpu.VMEM((NBUF, K, D), jnp.float32),
            pltpu.VMEM((ZROWS, D), jnp.float32),
            pltpu.VMEM_SHARED((N, D), jnp.float32),
        ] + [pltpu.SemaphoreType.DMA] * 12,
    )(x, packed)


NB = 10
BR = N // NB


def _mlp_phases(x_ref, agg_ref, w1, b1, g1, be1, w2, b2, g2, be2, wh, bh,
                out_ref, h1buf, h2buf, s1, s2, s3, s4, sc1, sh1, sc2, sh2):
    p = pl.program_id(0)
    i = pl.program_id(1)
    rows = pl.ds(i * BR, BR)

    @pl.when((p == 0) & (i == 0))
    def _init0():
        s1[...] = jnp.zeros_like(s1)
        s2[...] = jnp.zeros_like(s2)

    @pl.when(p == 0)
    def _phase0():
        z = x_ref[...] + agg_ref[0] + agg_ref[1]
        h = jnp.dot(z, w1[...], preferred_element_type=jnp.float32) + b1[...]
        h1buf[rows, :] = h
        s1[...] += jnp.sum(h, axis=0, keepdims=True)
        s2[...] += jnp.sum(h * h, axis=0, keepdims=True)

    @pl.when((p == 1) & (i == 0))
    def _stats1():
        m = s1[...] / N
        v = s2[...] / N - m * m
        sc = g1[...] * lax.rsqrt(v + BN_EPS)
        sc1[...] = sc
        sh1[...] = be1[...] - m * sc
        s3[...] = jnp.zeros_like(s3)
        s4[...] = jnp.zeros_like(s4)

    @pl.when(p == 1)
    def _phase1():
        a = jnp.maximum(h1buf[rows, :] * sc1[...] + sh1[...], 0.0)
        h = jnp.dot(a, w2[...], preferred_element_type=jnp.float32) + b2[...]
        h2buf[rows, :] = h
        s3[...] += jnp.sum(h, axis=0, keepdims=True)
        s4[...] += jnp.sum(h * h, axis=0, keepdims=True)

    @pl.when((p == 2) & (i == 0))
    def _stats2():
        m = s3[...] / N
        v = s4[...] / N - m * m
        sc = g2[...] * lax.rsqrt(v + BN_EPS)
        sc2[...] = sc
        sh2[...] = be2[...] - m * sc

    @pl.when(p == 2)
    def _phase2():
        a = jnp.maximum(h2buf[rows, :] * sc2[...] + sh2[...], 0.0)
        if wh is None:
            out_ref[...] = a
        else:
            out_ref[...] = (jnp.dot(a, wh[...],
                                    preferred_element_type=jnp.float32)
                            + bh[...])


def _mlp_body_plain(x_ref, agg_ref, w1, b1, g1, be1, w2, b2, g2, be2,
                    out_ref, h1buf, h2buf, s1, s2, s3, s4,
                    sc1, sh1, sc2, sh2):
    _mlp_phases(x_ref, agg_ref, w1, b1, g1, be1, w2, b2, g2, be2, None, None,
                out_ref, h1buf, h2buf, s1, s2, s3, s4, sc1, sh1, sc2, sh2)


def _mlp_body_heads(x_ref, agg_ref, w1, b1, g1, be1, w2, b2, g2, be2, wh, bh,
                    out_ref, h1buf, h2buf, s1, s2, s3, s4,
                    sc1, sh1, sc2, sh2):
    _mlp_phases(x_ref, agg_ref, w1, b1, g1, be1, w2, b2, g2, be2, wh, bh,
                out_ref, h1buf, h2buf, s1, s2, s3, s4, sc1, sh1, sc2, sh2)


def _row_spec(shape):
    return pl.BlockSpec(shape, lambda p, i: (jnp.where(p == 0, i, NB - 1), 0))


def _full_spec(shape):
    zeros = (0,) * len(shape)
    return pl.BlockSpec(shape, lambda p, i, z=zeros: z)


def _mlp_call(body, n_weights, out_w, operands):
    in_specs = [
        _row_spec((BR, D)),
        pl.BlockSpec((NC, BR, D),
                     lambda p, i: (0, jnp.where(p == 0, i, NB - 1), 0)),
    ] + [_full_spec(o.shape) for o in operands[2:]]
    return pl.pallas_call(
        body,
        grid=(3, NB),
        in_specs=in_specs,
        out_specs=pl.BlockSpec((BR, out_w),
                               lambda p, i: (jnp.where(p == 2, i, 0), 0)),
        out_shape=jax.ShapeDtypeStruct((N, out_w), jnp.float32),
        scratch_shapes=[
            pltpu.VMEM((N, D), jnp.float32),
            pltpu.VMEM((N, D), jnp.float32),
        ] + [pltpu.VMEM((1, D), jnp.float32)] * 8,
    )(*operands)


def _mlp1(x, agg, w1, b1, g1, be1, w2, b2, g2, be2):
    return _mlp_call(_mlp_body_plain, 8, D,
                     (x, agg, w1, b1, g1, be1, w2, b2, g2, be2))


def _mlp2(x, agg, w1, b1, g1, be1, w2, b2, g2, be2, wh, bh):
    return _mlp_call(_mlp_body_heads, 10, 2 * 64,
                     (x, agg, w1, b1, g1, be1, w2, b2, g2, be2, wh, bh))


def kernel(x, edge_index, W1a, b1a, g1a, be1a, W2a, b2a, g2a, be2a,
           W1b, b1b, g1b, be1b, W2b, b2b, g2b, be2b, Wmu, bmu, Wls, bls):
    src = edge_index[0].astype(jnp.int32)
    dst = edge_index[1].astype(jnp.int32)
    packed = src | (dst << 16)

    r = lambda v: v.reshape(1, -1)
    wh = jnp.concatenate([Wmu, Wls], axis=1)
    bh = jnp.concatenate([bmu, bls], axis=0).reshape(1, -1)

    agg1 = _sc_agg(x, packed)
    h1 = _mlp1(x, agg1, W1a, r(b1a), r(g1a), r(be1a),
               W2a, r(b2a), r(g2a), r(be2a))
    agg2 = _sc_agg(h1, packed)
    heads = _mlp2(h1, agg2, W1b, r(b1b), r(g1b), r(be1b),
                  W2b, r(b2b), r(g2b), r(be2b), wh, bh)
    return (heads[:, :64], heads[:, 64:])


# speedup vs baseline: 11.9608x; 1.0653x over previous
import jax
import jax.numpy as jnp
from jax import lax
from jax.experimental import pallas as pl
from jax.experimental.pallas import tpu as pltpu
from jax.experimental.pallas import tpu_sc as plsc

N = 10000
D = 128
E = 320000
NC = 2
NS = 16
NW = NC * NS
EPW = E // NW
K = 80
NCHUNK = EPW // K
RPS = 624
REM = N - NS * RPS
ZROWS = 48
NBUF = 4
PF = 3
BN_EPS = 1e-5


def _sc_agg_body(x_hbm, pk_hbm, out_hbm,
                 pk_v, src_v, dst_v, rows_v, zbuf_v, agg_sh,
                 gs0, gs1, gs2, gs3, ss0, ss1, ss2, ss3,
                 is0, is1, is2, is3):
    gsem = (gs0, gs1, gs2, gs3)
    ssem = (ss0, ss1, ss2, ss3)
    isem = (is0, is1, is2, is3)
    c = lax.axis_index("c")
    s = lax.axis_index("s")
    wid = c * NS + s
    base = wid * EPW

    z = jnp.zeros((16,), jnp.float32)

    def zrow(i, carry):
        def zcol(j, carry2):
            zbuf_v[i, pl.ds(j * 16, 16)] = z
            return carry2
        return lax.fori_loop(0, D // 16, zcol, carry)

    lax.fori_loop(0, ZROWS, zrow, 0)

    def zcopy(t, carry):
        pltpu.sync_copy(zbuf_v, agg_sh.at[pl.ds(s * RPS + t * ZROWS, ZROWS)])
        return carry

    lax.fori_loop(0, RPS // ZROWS, zcopy, 0)

    @pl.when(s == NS - 1)
    def _zero_rem():
        pltpu.sync_copy(zbuf_v.at[pl.ds(0, REM)],
                        agg_sh.at[pl.ds(NS * RPS, REM)])

    plsc.subcore_barrier()

    def fire_pk(j, b):
        pltpu.async_copy(pk_hbm.at[pl.ds(base + j * K, K)], pk_v.at[b],
                         isem[b])

    def wait_pk(b):
        pltpu.make_async_copy(pk_hbm.at[pl.ds(base, K)], pk_v.at[b],
                              isem[b]).wait()

    def decode(b):
        for u in range(K // 16):
            v = pk_v[b, pl.ds(u * 16, 16)]
            src_v[b, pl.ds(u * 16, 16)] = v & 0xFFFF
            dst_v[b, pl.ds(u * 16, 16)] = lax.shift_right_logical(v, 16)

    def fire_gather(b):
        pltpu.async_copy(x_hbm.at[src_v.at[b]], rows_v.at[b], gsem[b])

    def wait_gather(b):
        pltpu.make_async_copy(x_hbm.at[src_v.at[b]], rows_v.at[b],
                              gsem[b]).wait()

    def fire_scatter(b):
        pltpu.async_copy(rows_v.at[b], agg_sh.at[dst_v.at[b]], ssem[b],
                         add=True)

    def wait_scatter(b):
        pltpu.make_async_copy(rows_v.at[b], agg_sh.at[dst_v.at[b]],
                              ssem[b]).wait()

    for jj in range(PF):
        fire_pk(jj, jj)
    for jj in range(PF):
        wait_pk(jj)
        decode(jj)
        fire_gather(jj)

    def quad(t, carry):
        for r in range(NBUF):
            j = NBUF * t + r
            wait_gather(r)
            fire_scatter(r)
            fb = (r + PF) % NBUF
            fj = j + PF

            @pl.when(fj < NCHUNK)
            def _prefetch(fb=fb, fj=fj):
                fire_pk(fj, fb)

                @pl.when(fj >= NBUF)
                def _free_bufs():
                    wait_scatter(fb)

                wait_pk(fb)
                decode(fb)
                fire_gather(fb)

        return carry

    lax.fori_loop(0, NCHUNK // NBUF, quad, 0)
    wait_gather(0)
    fire_scatter(0)
    wait_scatter(1)
    wait_scatter(2)
    wait_scatter(3)
    wait_scatter(0)
    plsc.subcore_barrier()

    pltpu.sync_copy(agg_sh.at[pl.ds(s * RPS, RPS)],
                    out_hbm.at[c, pl.ds(s * RPS, RPS)])

    @pl.when(s == NS - 1)
    def _write_rem():
        pltpu.sync_copy(agg_sh.at[pl.ds(NS * RPS, REM)],
                        out_hbm.at[c, pl.ds(NS * RPS, REM)])


def _sc_agg(x, packed):
    mesh = plsc.VectorSubcoreMesh(core_axis_name="c", subcore_axis_name="s")
    return pl.kernel(
        _sc_agg_body,
        out_type=jax.ShapeDtypeStruct((NC, N, D), jnp.float32),
        mesh=mesh,
        scratch_types=[
            pltpu.VMEM((NBUF, K), jnp.int32),
            pltpu.VMEM((NBUF, K), jnp.int32),
            pltpu.VMEM((NBUF, K), jnp.int32),
            pltpu.VMEM((NBUF, K, D), jnp.float32),
            pltpu.VMEM((ZROWS, D), jnp.float32),
            pltpu.VMEM_SHARED((N, D), jnp.float32),
        ] + [pltpu.SemaphoreType.DMA] * 12,
    )(x, packed)


NB = 10
BR = N // NB
PKB = 32000


def _pack_body(ei_ref, out_ref):
    out_ref[...] = ei_ref[0, :] | (ei_ref[1, :] << 16)


def _pack_idx(edge_index):
    return pl.pallas_call(
        _pack_body,
        out_shape=jax.ShapeDtypeStruct((E,), jnp.int32),
    )(edge_index)


def _mlp_phases(x_ref, agg_ref, w1, b1, g1, be1, w2, b2, g2, be2, wh, bh,
                out_ref, h1buf, h2buf, s1, s2, s3, s4, sc1, sh1, sc2, sh2):
    p = pl.program_id(0)
    i = pl.program_id(1)
    rows = pl.ds(i * BR, BR)

    @pl.when((p == 0) & (i == 0))
    def _init0():
        s1[...] = jnp.zeros_like(s1)
        s2[...] = jnp.zeros_like(s2)

    @pl.when(p == 0)
    def _phase0():
        z = x_ref[...] + agg_ref[0] + agg_ref[1]
        h = jnp.dot(z, w1[...], preferred_element_type=jnp.float32) + b1[...]
        h1buf[rows, :] = h
        s1[...] += jnp.sum(h, axis=0, keepdims=True)
        s2[...] += jnp.sum(h * h, axis=0, keepdims=True)

    @pl.when((p == 1) & (i == 0))
    def _stats1():
        m = s1[...] / N
        v = s2[...] / N - m * m
        sc = g1[...] * lax.rsqrt(v + BN_EPS)
        sc1[...] = sc
        sh1[...] = be1[...] - m * sc
        s3[...] = jnp.zeros_like(s3)
        s4[...] = jnp.zeros_like(s4)

    @pl.when(p == 1)
    def _phase1():
        a = jnp.maximum(h1buf[rows, :] * sc1[...] + sh1[...], 0.0)
        h = jnp.dot(a, w2[...], preferred_element_type=jnp.float32) + b2[...]
        h2buf[rows, :] = h
        s3[...] += jnp.sum(h, axis=0, keepdims=True)
        s4[...] += jnp.sum(h * h, axis=0, keepdims=True)

    @pl.when((p == 2) & (i == 0))
    def _stats2():
        m = s3[...] / N
        v = s4[...] / N - m * m
        sc = g2[...] * lax.rsqrt(v + BN_EPS)
        sc2[...] = sc
        sh2[...] = be2[...] - m * sc

    @pl.when(p == 2)
    def _phase2():
        a = jnp.maximum(h2buf[rows, :] * sc2[...] + sh2[...], 0.0)
        if wh is None:
            out_ref[...] = a
        else:
            hh = (jnp.dot(a, wh[...], preferred_element_type=jnp.float32)
                  + bh[...])
            mu_ref, ls_ref = out_ref
            mu_ref[...] = hh[:, :64]
            ls_ref[...] = hh[:, 64:]


def _mlp_body_plain(x_ref, agg_ref, w1, b1, g1, be1, w2, b2, g2, be2,
                    out_ref, h1buf, h2buf, s1, s2, s3, s4,
                    sc1, sh1, sc2, sh2):
    _mlp_phases(x_ref, agg_ref, w1, b1, g1, be1, w2, b2, g2, be2, None, None,
                out_ref, h1buf, h2buf, s1, s2, s3, s4, sc1, sh1, sc2, sh2)


def _mlp_body_heads(x_ref, agg_ref, w1, b1, g1, be1, w2, b2, g2, be2, wh, bh,
                    mu_ref, ls_ref, h1buf, h2buf, s1, s2, s3, s4,
                    sc1, sh1, sc2, sh2):
    _mlp_phases(x_ref, agg_ref, w1, b1, g1, be1, w2, b2, g2, be2, wh, bh,
                (mu_ref, ls_ref), h1buf, h2buf, s1, s2, s3, s4,
                sc1, sh1, sc2, sh2)


def _row_spec(shape):
    return pl.BlockSpec(shape, lambda p, i: (jnp.where(p == 0, i, NB - 1), 0))


def _full_spec(shape):
    zeros = (0,) * len(shape)
    return pl.BlockSpec(shape, lambda p, i, z=zeros: z)


def _mlp_call(body, out_specs, out_shape, operands):
    in_specs = [
        _row_spec((BR, D)),
        pl.BlockSpec((NC, BR, D),
                     lambda p, i: (0, jnp.where(p == 0, i, NB - 1), 0)),
    ] + [_full_spec(o.shape) for o in operands[2:]]
    return pl.pallas_call(
        body,
        grid=(3, NB),
        in_specs=in_specs,
        out_specs=out_specs,
        out_shape=out_shape,
        scratch_shapes=[
            pltpu.VMEM((N, D), jnp.float32),
            pltpu.VMEM((N, D), jnp.float32),
        ] + [pltpu.VMEM((1, D), jnp.float32)] * 8,
    )(*operands)


def _mlp1(x, agg, w1, b1, g1, be1, w2, b2, g2, be2):
    return _mlp_call(
        _mlp_body_plain,
        pl.BlockSpec((BR, D), lambda p, i: (jnp.where(p == 2, i, 0), 0)),
        jax.ShapeDtypeStruct((N, D), jnp.float32),
        (x, agg, w1, b1, g1, be1, w2, b2, g2, be2))


def _mlp2(x, agg, w1, b1, g1, be1, w2, b2, g2, be2, wh, bh):
    hspec = pl.BlockSpec((BR, 64), lambda p, i: (jnp.where(p == 2, i, 0), 0))
    return _mlp_call(
        _mlp_body_heads,
        (hspec, hspec),
        (jax.ShapeDtypeStruct((N, 64), jnp.float32),
         jax.ShapeDtypeStruct((N, 64), jnp.float32)),
        (x, agg, w1, b1, g1, be1, w2, b2, g2, be2, wh, bh))


def kernel(x, edge_index, W1a, b1a, g1a, be1a, W2a, b2a, g2a, be2a,
           W1b, b1b, g1b, be1b, W2b, b2b, g2b, be2b, Wmu, bmu, Wls, bls):
    packed = _pack_idx(edge_index.astype(jnp.int32))

    r = lambda v: v.reshape(1, -1)
    wh = jnp.concatenate([Wmu, Wls], axis=1)
    bh = jnp.concatenate([bmu, bls], axis=0).reshape(1, -1)

    agg1 = _sc_agg(x, packed)
    h1 = _mlp1(x, agg1, W1a, r(b1a), r(g1a), r(be1a),
               W2a, r(b2a), r(g2a), r(be2a))
    agg2 = _sc_agg(h1, packed)
    mu, ls = _mlp2(h1, agg2, W1b, r(b1b), r(g1b), r(be1b),
                   W2b, r(b2b), r(g2b), r(be2b), wh, bh)
    return (mu, ls)
